# Initial kernel scaffold; baseline (speedup 1.0000x reference)
import jax
import jax.numpy as jnp
from jax.experimental import pallas as pl


def kernel(intervention_label, permutation, masks):
    raise NotImplementedError("write your pallas kernel here")


# speedup vs baseline: 3.6640x; 3.6640x over previous
import functools

import jax
import jax.numpy as jnp
from jax import lax
from jax.experimental import pallas as pl
from jax.experimental.pallas import tpu as pltpu
from jax.experimental.pallas import tpu_sc as plsc

DIM_Z = 64
N_INT = 65
PERM_PAD = 80
NC, NS = 2, 16
NW = NC * NS
LANES = 16
CHUNK = 128


def _make_sc_lookup(batch: int):
    bpw = batch // NW
    n_chunks = bpw // CHUNK
    mesh = plsc.VectorSubcoreMesh(core_axis_name="c", subcore_axis_name="s")

    @functools.partial(
        pl.kernel,
        mesh=mesh,
        out_type=jax.ShapeDtypeStruct((batch, DIM_Z), jnp.bool_),
        scratch_types=[
            pltpu.VMEM((bpw,), jnp.int32),
            pltpu.VMEM((PERM_PAD,), jnp.int32),
            pltpu.VMEM((n_chunks, CHUNK), jnp.int32),
            pltpu.VMEM((bpw, DIM_Z), jnp.bool_),
            pltpu.SemaphoreType.DMA,
        ],
        compiler_params=pltpu.CompilerParams(
            needs_layout_passes=False, use_tc_tiling_on_sc=False),
    )
    def sc_lookup(labels_hbm, perm_hbm, masks_hbm, out_hbm,
                  labels_v, perm_v, idx_v, rows_v, sem):
        wid = lax.axis_index("s") * NC + lax.axis_index("c")
        base = wid * bpw
        pltpu.sync_copy(labels_hbm.at[pl.ds(base, bpw)], labels_v)
        pltpu.sync_copy(perm_hbm, perm_v)
        for i in range(bpw // LANES):
            lbl = labels_v[pl.ds(i * LANES, LANES)]
            iv = plsc.load_gather(perm_v, [lbl])
            idx_v[i // (CHUNK // LANES),
                  pl.ds((i % (CHUNK // LANES)) * LANES, LANES)] = iv
        copies = [
            pltpu.async_copy(masks_hbm.at[idx_v.at[j]],
                             rows_v.at[pl.ds(j * CHUNK, CHUNK)], sem)
            for j in range(n_chunks)
        ]
        for c in copies:
            c.wait()
        pltpu.sync_copy(rows_v, out_hbm.at[pl.ds(base, bpw)])

    return sc_lookup


def kernel(intervention_label, permutation, masks):
    batch = intervention_label.shape[0]
    perm_padded = jnp.concatenate(
        [permutation, jnp.zeros((PERM_PAD - N_INT,), jnp.int32)])
    return _make_sc_lookup(batch)(intervention_label, perm_padded, masks)
